# Initial kernel scaffold; baseline (speedup 1.0000x reference)
import functools

import jax
import jax.numpy as jnp
from jax import lax
from jax.experimental import pallas as pl
from jax.experimental.pallas import tpu as pltpu
from jax.experimental.pallas import tpu_sc as plsc


def _make_kernel(B, D, R):
    info = plsc.get_sparse_core_info()
    NC, NS, L = info.num_cores, info.num_subcores, info.num_lanes
    NW = NC * NS
    b_per_w = B // NW
    CH = 128
    n_chunks = b_per_w // CH
    NBUF = 2

    mesh = plsc.VectorSubcoreMesh(core_axis_name="c", subcore_axis_name="s")

    scratch = []
    for _ in range(NBUF):
        scratch += [
            pltpu.VMEM((CH, D), jnp.float32),
            pltpu.VMEM((CH, D), jnp.float32),
            pltpu.VMEM((CH, D), jnp.float32),
        ]
    scratch += [
        pltpu.VMEM((n_chunks, CH), jnp.int32),
        pltpu.VMEM((CH,), jnp.float32),
        pltpu.VMEM((16,), jnp.float32),
    ]
    scratch += [pltpu.SemaphoreType.DMA, pltpu.SemaphoreType.DMA]

    @functools.partial(
        pl.kernel,
        mesh=mesh,
        out_type=jax.ShapeDtypeStruct((B,), jnp.float32),
        scratch_types=scratch,
    )
    def k(s_hbm, r_hbm, o_hbm, rel_hbm, scale_hbm, out_hbm,
          w0, s0, o0, w1, s1, o1, idx_v, out_v, scale_v, sem0, sem1):
        wid = lax.axis_index("s") * NC + lax.axis_index("c")
        base_chunk = wid * n_chunks

        bufs = ((w0, s0, o0, sem0), (w1, s1, o1, sem1))

        pltpu.sync_copy(r_hbm.at[pl.ds(base_chunk, n_chunks)], idx_v)
        pltpu.sync_copy(scale_hbm, scale_v)
        sv = scale_v[...]

        def start(c, slot):
            w_b, s_b, o_b, sem = bufs[slot]
            row0 = (base_chunk + c) * CH
            hw = pltpu.async_copy(rel_hbm.at[idx_v.at[c]], w_b, sem)
            hs = pltpu.async_copy(s_hbm.at[pl.ds(row0, CH)], s_b, sem)
            ho = pltpu.async_copy(o_hbm.at[pl.ds(row0, CH)], o_b, sem)
            return (hw, hs, ho)

        rows0 = lax.iota(jnp.int32, L)
        handles = [None] * NBUF
        handles[0] = start(0, 0)

        for c in range(n_chunks):
            slot = c % NBUF
            if c + 1 < n_chunks:
                handles[(c + 1) % NBUF] = start(c + 1, (c + 1) % NBUF)
            for h in handles[slot]:
                h.wait()
            w_b, s_b, o_b, _ = bufs[slot]
            for g in range(CH // L):
                rows = rows0 + g * L

                def body(d, acc):
                    cold = jnp.full((L,), d, jnp.int32)
                    sx = plsc.load_gather(s_b, [rows, cold])
                    wx = plsc.load_gather(w_b, [rows, cold])
                    ox = plsc.load_gather(o_b, [rows, cold])
                    return acc + sx * wx * ox

                acc = lax.fori_loop(0, D, body, jnp.zeros((L,), jnp.float32))
                out_v[pl.ds(g * L, L)] = acc * sv
            pltpu.sync_copy(out_v, out_hbm.at[pl.ds((base_chunk + c) * CH, CH)])

    return k


def kernel(s, r, o, rel, scale):
    B, D = s.shape
    R = rel.shape[0]
    r2 = r.astype(jnp.int32).reshape(B // 128, 128)
    scale_splat = jnp.full((16,), scale, dtype=jnp.float32)
    k = _make_kernel(B, D, R)
    return k(s, r2, o, rel, scale_splat)


# speedup vs baseline: 1.0110x; 1.0110x over previous
import functools

import jax
import jax.numpy as jnp
from jax import lax
from jax.experimental import pallas as pl
from jax.experimental.pallas import tpu as pltpu
from jax.experimental.pallas import tpu_sc as plsc


def _make_kernel(B, D, R):
    info = plsc.get_sparse_core_info()
    NC, NS, L = info.num_cores, info.num_subcores, info.num_lanes
    NW = NC * NS
    b_per_w = B // NW
    CH = 128
    n_chunks = b_per_w // CH
    GROUP = 16

    mesh = plsc.VectorSubcoreMesh(core_axis_name="c", subcore_axis_name="s")

    scratch = []
    for _ in range(2):
        scratch += [
            pltpu.VMEM((CH, D), jnp.float32),
            pltpu.VMEM((CH, D), jnp.float32),
            pltpu.VMEM((CH, D), jnp.float32),
        ]
    scratch += [
        pltpu.VMEM((n_chunks, CH), jnp.int32),
        pltpu.VMEM((CH,), jnp.float32),
        pltpu.VMEM((16,), jnp.float32),
    ]
    scratch += [pltpu.SemaphoreType.DMA, pltpu.SemaphoreType.DMA]

    @functools.partial(
        pl.kernel,
        mesh=mesh,
        out_type=jax.ShapeDtypeStruct((B,), jnp.float32),
        scratch_types=scratch,
        compiler_params=pltpu.CompilerParams(needs_layout_passes=False),
    )
    def k(s_hbm, r_hbm, o_hbm, rel_hbm, scale_hbm, out_hbm,
          w0, s0, o0, w1, s1, o1, idx_v, out_v, scale_v, sem0, sem1):
        wid = lax.axis_index("s") * NC + lax.axis_index("c")
        base_chunk = wid * n_chunks

        bufs = ((w0, s0, o0, sem0), (w1, s1, o1, sem1))

        pltpu.sync_copy(r_hbm.at[pl.ds(base_chunk, n_chunks)], idx_v)
        pltpu.sync_copy(scale_hbm, scale_v)
        sv = scale_v[...][0]

        def start(c, slot):
            w_b, s_b, o_b, sem = bufs[slot]
            row0 = (base_chunk + c) * CH
            hw = pltpu.async_copy(rel_hbm.at[idx_v.at[c]], w_b, sem)
            hs = pltpu.async_copy(s_hbm.at[pl.ds(row0, CH)], s_b, sem)
            ho = pltpu.async_copy(o_hbm.at[pl.ds(row0, CH)], o_b, sem)
            return (hw, hs, ho)

        handles = [None, None]
        handles[0] = start(0, 0)

        for c in range(n_chunks):
            slot = c % 2
            if c + 1 < n_chunks:
                handles[(c + 1) % 2] = start(c + 1, (c + 1) % 2)
            for h in handles[slot]:
                h.wait()
            w_b, s_b, o_b, _ = bufs[slot]

            def group_body(g, carry, w_b=w_b, s_b=s_b, o_b=o_b):
                row0 = g * GROUP
                lane = lax.iota(jnp.int32, L)
                res = jnp.zeros((L,), jnp.float32)
                for rr in range(GROUP):
                    row = row0 + rr
                    acc = jnp.zeros((L,), jnp.float32)
                    for j in range(D // L):
                        sx = s_b[row, pl.ds(j * L, L)]
                        wx = w_b[row, pl.ds(j * L, L)]
                        ox = o_b[row, pl.ds(j * L, L)]
                        acc = acc + sx * wx * ox
                    res = jnp.where(lane == rr, jnp.sum(acc) * sv, res)
                out_v[pl.ds(row0, L)] = res
                return carry

            lax.fori_loop(0, CH // GROUP, group_body, 0)
            pltpu.sync_copy(out_v, out_hbm.at[pl.ds((base_chunk + c) * CH, CH)])

    return k


def kernel(s, r, o, rel, scale):
    B, D = s.shape
    R = rel.shape[0]
    r2 = r.astype(jnp.int32).reshape(B // 128, 128)
    scale_splat = jnp.full((16,), scale, dtype=jnp.float32)
    k = _make_kernel(B, D, R)
    return k(s, r2, o, rel, scale_splat)
